# Initial kernel scaffold; baseline (speedup 1.0000x reference)
import functools

import jax
import jax.numpy as jnp
from jax import lax
from jax.experimental import pallas as pl
from jax.experimental.pallas import tpu as pltpu
from jax.experimental.pallas import tpu_sc as plsc

_NUM_FIELDS = 26
_VOCAB = 100000
_EMB_DIM = 32
_BATCH = 16384

_NC = 2
_NS = 16
_NW = _NC * _NS
_LANES = 16

_B_PER_W = _BATCH // _NW
_CH = 64
_N_CHUNKS = _B_PER_W // _CH
_IDX_PER_CH = _CH * _NUM_FIELDS
_IDX_MINOR = 128
_N_GATHER = _IDX_PER_CH // _IDX_MINOR
_PERIOD = 208


def _sc_kernel(attrs_hbm, tables_hbm, out_hbm,
               attrs_v, idx_v, rows_v, outb_v, offs_v, sem):
    wid = lax.axis_index("s") * _NC + lax.axis_index("c")

    for j in range(_PERIOD // _LANES):
        lane = lax.iota(jnp.int32, _LANES) + (j * _LANES)
        offs_v[pl.ds(j * _LANES, _LANES)] = (lane % _NUM_FIELDS) * _VOCAB

    def chunk_body(ci, carry):
        b0 = wid * _B_PER_W + ci * _CH
        pltpu.sync_copy(attrs_hbm.at[pl.ds(b0 * _NUM_FIELDS, _IDX_PER_CH)],
                        attrs_v)
        for j in range(_IDX_PER_CH // _LANES):
            col = j * _LANES
            v = attrs_v[pl.ds(col, _LANES)] + offs_v[pl.ds(col % _PERIOD,
                                                           _LANES)]
            idx_v[col // _IDX_MINOR, pl.ds(col % _IDX_MINOR, _LANES)] = v
        cps = []
        for r in range(_N_GATHER):
            cps.append(pltpu.async_copy(
                tables_hbm.at[idx_v.at[r]],
                rows_v.at[pl.ds(r * _IDX_MINOR, _IDX_MINOR)],
                sem))
        for cp in cps:
            cp.wait()

        def acc_body(bl, c):
            row0 = bl * _NUM_FIELDS
            a0 = rows_v[row0, pl.ds(0, _LANES)]
            a1 = rows_v[row0, pl.ds(_LANES, _LANES)]
            for i in range(1, _NUM_FIELDS):
                a0 = a0 + rows_v[row0 + i, pl.ds(0, _LANES)]
                a1 = a1 + rows_v[row0 + i, pl.ds(_LANES, _LANES)]
            outb_v[bl, pl.ds(0, _LANES)] = a0
            outb_v[bl, pl.ds(_LANES, _LANES)] = a1
            return c

        lax.fori_loop(0, _CH, acc_body, 0)
        pltpu.sync_copy(outb_v, out_hbm.at[pl.ds(b0, _CH)])
        return carry

    lax.fori_loop(0, _N_CHUNKS, chunk_body, 0)


@jax.jit
def kernel(attrs, tables):
    attrs_flat = attrs.astype(jnp.int32).reshape(-1)
    tables_flat = tables.reshape(_NUM_FIELDS * _VOCAB, _EMB_DIM)
    run = functools.partial(
        pl.kernel,
        mesh=plsc.VectorSubcoreMesh(core_axis_name="c", subcore_axis_name="s"),
        out_type=jax.ShapeDtypeStruct((_BATCH, _EMB_DIM), jnp.float32),
        scratch_types=[
            pltpu.VMEM((_IDX_PER_CH,), jnp.int32),
            pltpu.VMEM((_N_GATHER, _IDX_MINOR), jnp.int32),
            pltpu.VMEM((_IDX_PER_CH, _EMB_DIM), jnp.float32),
            pltpu.VMEM((_CH, _EMB_DIM), jnp.float32),
            pltpu.VMEM((_PERIOD,), jnp.int32),
            pltpu.SemaphoreType.DMA,
        ],
    )(_sc_kernel)
    return run(attrs_flat, tables_flat)


# speedup vs baseline: 1.1807x; 1.1807x over previous
import functools

import jax
import jax.numpy as jnp
from jax import lax
from jax.experimental import pallas as pl
from jax.experimental.pallas import tpu as pltpu
from jax.experimental.pallas import tpu_sc as plsc

_NUM_FIELDS = 26
_VOCAB = 100000
_EMB_DIM = 32
_BATCH = 16384

_NC = 2
_NS = 16
_NW = _NC * _NS
_LANES = 16

_B_PER_W = _BATCH // _NW
_CH = 64
_N_CHUNKS = _B_PER_W // _CH
_IDX_PER_CH = _CH * _NUM_FIELDS
_IDX_MINOR = 128
_N_GATHER = _IDX_PER_CH // _IDX_MINOR
_PERIOD = 208


def _sc_kernel(attrs_hbm, tables_hbm, out_hbm,
               attrs_v, idx_v, rows_v, outb_v, offs_v, sem):
    wid = lax.axis_index("s") * _NC + lax.axis_index("c")

    for j in range(_PERIOD // _LANES):
        lane = lax.iota(jnp.int32, _LANES) + (j * _LANES)
        offs_v[pl.ds(j * _LANES, _LANES)] = (lane % _NUM_FIELDS) * _VOCAB

    def chunk_body(ci, carry):
        b0 = wid * _B_PER_W + ci * _CH
        pltpu.sync_copy(attrs_hbm.at[pl.ds(b0 * _NUM_FIELDS, _IDX_PER_CH)],
                        attrs_v)
        for j in range(_IDX_PER_CH // _LANES):
            col = j * _LANES
            v = attrs_v[pl.ds(col, _LANES)] + offs_v[pl.ds(col % _PERIOD,
                                                           _LANES)]
            idx_v[col // _IDX_MINOR, pl.ds(col % _IDX_MINOR, _LANES)] = v
        cps = []
        for r in range(_N_GATHER):
            cps.append(pltpu.async_copy(
                tables_hbm.at[idx_v.at[r]],
                rows_v.at[pl.ds(r * _IDX_MINOR, _IDX_MINOR)],
                sem))
        for cp in cps:
            cp.wait()

        def acc_body(bl, c):
            row0 = bl * _NUM_FIELDS
            a0 = rows_v[row0, pl.ds(0, _LANES)]
            a1 = rows_v[row0, pl.ds(_LANES, _LANES)]
            for i in range(1, _NUM_FIELDS):
                a0 = a0 + rows_v[row0 + i, pl.ds(0, _LANES)]
                a1 = a1 + rows_v[row0 + i, pl.ds(_LANES, _LANES)]
            outb_v[bl, pl.ds(0, _LANES)] = a0
            outb_v[bl, pl.ds(_LANES, _LANES)] = a1
            return c

        lax.fori_loop(0, _CH, acc_body, 0)
        pltpu.sync_copy(outb_v, out_hbm.at[pl.ds(b0, _CH)])
        return carry

    lax.fori_loop(0, _N_CHUNKS, chunk_body, 0)


@jax.jit
def kernel(attrs, tables):
    attrs_flat = attrs.astype(jnp.int32).reshape(-1)
    tables_flat = tables.reshape(_NUM_FIELDS * _VOCAB, _EMB_DIM)
    run = functools.partial(
        pl.kernel,
        mesh=plsc.VectorSubcoreMesh(core_axis_name="c", subcore_axis_name="s"),
        compiler_params=pltpu.CompilerParams(use_tc_tiling_on_sc=False),
        out_type=jax.ShapeDtypeStruct((_BATCH, _EMB_DIM), jnp.float32),
        scratch_types=[
            pltpu.VMEM((_IDX_PER_CH,), jnp.int32),
            pltpu.VMEM((_N_GATHER, _IDX_MINOR), jnp.int32),
            pltpu.VMEM((_IDX_PER_CH, _EMB_DIM), jnp.float32),
            pltpu.VMEM((_CH, _EMB_DIM), jnp.float32),
            pltpu.VMEM((_PERIOD,), jnp.int32),
            pltpu.SemaphoreType.DMA,
        ],
    )(_sc_kernel)
    return run(attrs_flat, tables_flat)


# speedup vs baseline: 2.8643x; 2.4260x over previous
import functools

import jax
import jax.numpy as jnp
from jax import lax
from jax.experimental import pallas as pl
from jax.experimental.pallas import tpu as pltpu
from jax.experimental.pallas import tpu_sc as plsc

_NUM_FIELDS = 26
_VOCAB = 100000
_EMB_DIM = 32
_BATCH = 16384
_LANES = 16
_COL_CH = 2048
_N_CC = _BATCH // _COL_CH


def _sc_kernel(attrsT_hbm, tablesT_hbm, outT_hbm, row_v, col_v, acc_v):
    d = lax.axis_index("c") * 16 + lax.axis_index("s")

    for i in range(_NUM_FIELDS):
        pltpu.sync_copy(tablesT_hbm.at[i, d], row_v)
        for cc in range(_N_CC):
            b0 = cc * _COL_CH
            pltpu.sync_copy(attrsT_hbm.at[i, pl.ds(b0, _COL_CH)], col_v)

            def group_body(g, carry, *, base=b0, first=(i == 0)):
                off = g * _LANES
                v16 = col_v[pl.ds(off, _LANES)]
                val = plsc.load_gather(row_v, [v16])
                if first:
                    acc_v[pl.ds(base + off, _LANES)] = val
                else:
                    plsc.addupdate(acc_v.at[pl.ds(base + off, _LANES)], val)
                return carry

            lax.fori_loop(0, _COL_CH // _LANES, group_body, 0)

    pltpu.sync_copy(acc_v, outT_hbm.at[d])


@jax.jit
def kernel(attrs, tables):
    attrsT = attrs.astype(jnp.int32).T
    tablesT = jnp.transpose(tables, (0, 2, 1))
    run = functools.partial(
        pl.kernel,
        mesh=plsc.VectorSubcoreMesh(core_axis_name="c", subcore_axis_name="s"),
        compiler_params=pltpu.CompilerParams(needs_layout_passes=False),
        out_type=jax.ShapeDtypeStruct((_EMB_DIM, _BATCH), jnp.float32),
        scratch_types=[
            pltpu.VMEM((_VOCAB,), jnp.float32),
            pltpu.VMEM((_COL_CH,), jnp.int32),
            pltpu.VMEM((_BATCH,), jnp.float32),
        ],
    )(_sc_kernel)
    outT = run(attrsT, tablesT)
    return outT.T


# speedup vs baseline: 6.0769x; 2.1216x over previous
import functools

import jax
import jax.numpy as jnp
from jax import lax
from jax.experimental import pallas as pl
from jax.experimental.pallas import tpu as pltpu
from jax.experimental.pallas import tpu_sc as plsc

_NUM_FIELDS = 26
_VOCAB = 100000
_EMB_DIM = 32
_BATCH = 16384
_LANES = 16
_COL_CH = 4096
_N_CC = _BATCH // _COL_CH


def _sc_kernel(attrsT_hbm, tablesT_hbm, outT_hbm,
               row_v, col_a, col_b, acc_v, sem_a, sem_b):
    d = lax.axis_index("c") * 16 + lax.axis_index("s")
    cols = (col_a, col_b)
    sems = (sem_a, sem_b)

    for i in range(_NUM_FIELDS):
        pltpu.sync_copy(tablesT_hbm.at[i, d], row_v)
        cps = {}
        cps[0] = pltpu.async_copy(
            attrsT_hbm.at[i, pl.ds(0, _COL_CH)], cols[0], sems[0])
        for cc in range(_N_CC):
            b0 = cc * _COL_CH
            if cc + 1 < _N_CC:
                cps[cc + 1] = pltpu.async_copy(
                    attrsT_hbm.at[i, pl.ds((cc + 1) * _COL_CH, _COL_CH)],
                    cols[(cc + 1) % 2], sems[(cc + 1) % 2])
            cps.pop(cc).wait()
            col_v = cols[cc % 2]

            @plsc.parallel_loop(0, _COL_CH // _LANES, unroll=8)
            def group_body(g, *, col_v=col_v, b0=b0, first=(i == 0)):
                off = g * _LANES
                v16 = col_v[pl.ds(off, _LANES)]
                val = plsc.load_gather(row_v, [v16])
                if first:
                    acc_v[pl.ds(b0 + off, _LANES)] = val
                else:
                    plsc.addupdate(acc_v.at[pl.ds(b0 + off, _LANES)], val)

    pltpu.sync_copy(acc_v, outT_hbm.at[d])


@jax.jit
def kernel(attrs, tables):
    attrsT = attrs.astype(jnp.int32).T
    tablesT = jnp.transpose(tables, (0, 2, 1))
    run = functools.partial(
        pl.kernel,
        mesh=plsc.VectorSubcoreMesh(core_axis_name="c", subcore_axis_name="s"),
        compiler_params=pltpu.CompilerParams(needs_layout_passes=False),
        out_type=jax.ShapeDtypeStruct((_EMB_DIM, _BATCH), jnp.float32),
        scratch_types=[
            pltpu.VMEM((_VOCAB,), jnp.float32),
            pltpu.VMEM((_COL_CH,), jnp.int32),
            pltpu.VMEM((_COL_CH,), jnp.int32),
            pltpu.VMEM((_BATCH,), jnp.float32),
            pltpu.SemaphoreType.DMA,
            pltpu.SemaphoreType.DMA,
        ],
    )(_sc_kernel)
    outT = run(attrsT, tablesT)
    return outT.T


# speedup vs baseline: 6.3159x; 1.0393x over previous
import functools

import jax
import jax.numpy as jnp
from jax import lax
from jax.experimental import pallas as pl
from jax.experimental.pallas import tpu as pltpu
from jax.experimental.pallas import tpu_sc as plsc

_NUM_FIELDS = 26
_VOCAB = 100000
_EMB_DIM = 32
_BATCH = 16384
_LANES = 16
_COL_CH = 4096
_N_CC = _BATCH // _COL_CH


def _sc_kernel(attrsT_hbm, tablesT_hbm, outT_hbm,
               row_v, col_a, col_b, acc_v, sem_a, sem_b):
    d = lax.axis_index("c") * 16 + lax.axis_index("s")
    cols = (col_a, col_b)
    sems = (sem_a, sem_b)

    @plsc.parallel_loop(0, _BATCH // _LANES, unroll=8)
    def zero_body(g):
        acc_v[pl.ds(g * _LANES, _LANES)] = jnp.zeros((_LANES,), jnp.float32)

    def field_body(i, carry):
        pltpu.sync_copy(tablesT_hbm.at[i, d], row_v)
        cps = {}
        cps[0] = pltpu.async_copy(
            attrsT_hbm.at[i, pl.ds(0, _COL_CH)], cols[0], sems[0])
        for cc in range(_N_CC):
            b0 = cc * _COL_CH
            if cc + 1 < _N_CC:
                cps[cc + 1] = pltpu.async_copy(
                    attrsT_hbm.at[i, pl.ds((cc + 1) * _COL_CH, _COL_CH)],
                    cols[(cc + 1) % 2], sems[(cc + 1) % 2])
            cps.pop(cc).wait()
            col_v = cols[cc % 2]

            @plsc.parallel_loop(0, _COL_CH // _LANES, unroll=16)
            def group_body(g, *, col_v=col_v, b0=b0):
                off = g * _LANES
                v16 = col_v[pl.ds(off, _LANES)]
                val = plsc.load_gather(row_v, [v16])
                plsc.addupdate(acc_v.at[pl.ds(b0 + off, _LANES)], val)

        return carry

    lax.fori_loop(0, _NUM_FIELDS, field_body, 0)
    pltpu.sync_copy(acc_v, outT_hbm.at[d])


@jax.jit
def kernel(attrs, tables):
    attrsT = attrs.astype(jnp.int32).T
    tablesT = jnp.transpose(tables, (0, 2, 1))
    run = functools.partial(
        pl.kernel,
        mesh=plsc.VectorSubcoreMesh(core_axis_name="c", subcore_axis_name="s"),
        compiler_params=pltpu.CompilerParams(needs_layout_passes=False),
        out_type=jax.ShapeDtypeStruct((_EMB_DIM, _BATCH), jnp.float32),
        scratch_types=[
            pltpu.VMEM((_VOCAB,), jnp.float32),
            pltpu.VMEM((_COL_CH,), jnp.int32),
            pltpu.VMEM((_COL_CH,), jnp.int32),
            pltpu.VMEM((_BATCH,), jnp.float32),
            pltpu.SemaphoreType.DMA,
            pltpu.SemaphoreType.DMA,
        ],
    )(_sc_kernel)
    outT = run(attrsT, tablesT)
    return outT.T


# speedup vs baseline: 6.5099x; 1.0307x over previous
import functools

import jax
import jax.numpy as jnp
from jax import lax
from jax.experimental import pallas as pl
from jax.experimental.pallas import tpu as pltpu
from jax.experimental.pallas import tpu_sc as plsc

_NUM_FIELDS = 26
_VOCAB = 100000
_EMB_DIM = 32
_BATCH = 16384
_LANES = 16
_COL_CH = 4096
_N_CC = _BATCH // _COL_CH


def _sc_kernel(attrsT_hbm, tablesT_hbm, outT_hbm,
               row_v, col_a, col_b, acc_v, sem_a, sem_b):
    d = lax.axis_index("c") * 16 + lax.axis_index("s")
    cols = (col_a, col_b)
    sems = (sem_a, sem_b)

    @plsc.parallel_loop(0, _BATCH // _LANES, unroll=8)
    def zero_body(g):
        acc_v[pl.ds(g * _LANES, _LANES)] = jnp.zeros((_LANES,), jnp.float32)

    def field_body(i, carry):
        cps = {}
        cps[0] = pltpu.async_copy(
            attrsT_hbm.at[i, pl.ds(0, _COL_CH)], cols[0], sems[0])
        pltpu.sync_copy(tablesT_hbm.at[i, d], row_v)
        for cc in range(_N_CC):
            b0 = cc * _COL_CH
            if cc + 1 < _N_CC:
                cps[cc + 1] = pltpu.async_copy(
                    attrsT_hbm.at[i, pl.ds((cc + 1) * _COL_CH, _COL_CH)],
                    cols[(cc + 1) % 2], sems[(cc + 1) % 2])
            cps.pop(cc).wait()
            col_v = cols[cc % 2]

            @plsc.parallel_loop(0, _COL_CH // _LANES, unroll=32)
            def group_body(g, *, col_v=col_v, b0=b0):
                off = g * _LANES
                v16 = col_v[pl.ds(off, _LANES)]
                val = plsc.load_gather(row_v, [v16])
                plsc.addupdate(acc_v.at[pl.ds(b0 + off, _LANES)], val)

        return carry

    lax.fori_loop(0, _NUM_FIELDS, field_body, 0)
    pltpu.sync_copy(acc_v, outT_hbm.at[d])


@jax.jit
def kernel(attrs, tables):
    attrsT = attrs.astype(jnp.int32).T
    tablesT = jnp.transpose(tables, (0, 2, 1))
    run = functools.partial(
        pl.kernel,
        mesh=plsc.VectorSubcoreMesh(core_axis_name="c", subcore_axis_name="s"),
        compiler_params=pltpu.CompilerParams(needs_layout_passes=False),
        out_type=jax.ShapeDtypeStruct((_EMB_DIM, _BATCH), jnp.float32),
        scratch_types=[
            pltpu.VMEM((_VOCAB,), jnp.float32),
            pltpu.VMEM((_COL_CH,), jnp.int32),
            pltpu.VMEM((_COL_CH,), jnp.int32),
            pltpu.VMEM((_BATCH,), jnp.float32),
            pltpu.SemaphoreType.DMA,
            pltpu.SemaphoreType.DMA,
        ],
    )(_sc_kernel)
    outT = run(attrsT, tablesT)
    return outT.T
